# baseline (speedup 1.0000x reference)
import functools

import jax
import jax.numpy as jnp
from jax import lax
from jax.experimental import pallas as pl
from jax.experimental.pallas import tpu as pltpu
from jax.experimental.pallas import tpu_sc as plsc

N, E, D = 10000, 320000, 128
NC, NS = 2, 16
NW = NC * NS
CHUNK = 128
NCH = 80
SEG = 5
CPS = NCH // SEG
EPW = NCH * CHUNK
EPAD = NW * EPW
NPAD = 240
RPS = 624
RREM = N - NS * RPS


ZROWS = 640


def _sc_segment_partials(h, edges4, zeros):
    mesh = plsc.VectorSubcoreMesh(core_axis_name="c", subcore_axis_name="s")

    @functools.partial(
        pl.kernel,
        out_type=jax.ShapeDtypeStruct((NC, N, D), jnp.float32),
        mesh=mesh,
        scratch_types=[
            pltpu.VMEM((CPS, CHUNK), jnp.int32),
            pltpu.VMEM((CPS, CHUNK), jnp.int32),
            pltpu.VMEM((CPS, CHUNK), jnp.int32),
            pltpu.VMEM((CPS, CHUNK), jnp.int32),
            pltpu.VMEM((CHUNK, D), jnp.float32),
            pltpu.VMEM((CHUNK, D), jnp.float32),
            pltpu.VMEM_SHARED((N + NPAD, D), jnp.float32),
            pltpu.SemaphoreType.DMA,
            pltpu.SemaphoreType.DMA,
            pltpu.SemaphoreType.DMA,
            pltpu.SemaphoreType.DMA,
            pltpu.SemaphoreType.DMA,
            pltpu.SemaphoreType.DMA,
        ],
    )
    def k(h_hbm, e_hbm, zero_hbm, out_hbm,
          is0, id0, is1, id1, buf0, buf1, agg_sh,
          sem0, sem1, semi0, semi1, sems0, sems1):
        c = lax.axis_index("c")
        s = lax.axis_index("s")
        wid = c * NS + s

        def idx_copies(q, isb, idb, semi):
            qs = pl.ds(q * CPS, CPS)
            return (
                pltpu.make_async_copy(e_hbm.at[0].at[wid].at[qs], isb, semi),
                pltpu.make_async_copy(e_hbm.at[1].at[wid].at[qs], idb, semi),
            )

        def load_idx(q, isb, idb, semi):
            for cp in idx_copies(q, isb, idb, semi):
                cp.start()

        def wait_idx(q, isb, idb, semi):
            for cp in idx_copies(q, isb, idb, semi):
                cp.wait()

        load_idx(0, is0, id0, semi0)

        pltpu.sync_copy(zero_hbm.at[pl.ds(0, RPS)],
                        agg_sh.at[pl.ds(s * RPS, RPS)])

        @pl.when(s == NS - 1)
        def _():
            pltpu.sync_copy(zero_hbm.at[pl.ds(0, RREM + NPAD)],
                            agg_sh.at[pl.ds(NS * RPS, RREM + NPAD)])

        plsc.subcore_barrier()

        def g_copy(i, isb, buf, sem):
            return pltpu.make_async_copy(h_hbm.at[isb.at[i]], buf, sem)

        def s_copy(i, idb, buf, sem):
            return pltpu.async_copy(buf, agg_sh.at[idb.at[i]], sem, add=True)

        def s_wait(i, idb, buf, sem):
            pltpu.make_async_copy(buf, agg_sh.at[idb.at[i]], sem).wait()

        banks = ((is0, id0, semi0), (is1, id1, semi1))
        for q in range(SEG):
            isb, idb, semi = banks[q % 2]
            wait_idx(q, isb, idb, semi)
            if q + 1 < SEG:
                load_idx(q + 1, *banks[(q + 1) % 2])
            if q > 0:
                pidb = banks[(q - 1) % 2][1]
                s_wait(CPS - 2, pidb, buf0, sems0)
                s_wait(CPS - 1, pidb, buf1, sems1)

            g_copy(0, isb, buf0, sem0).start()
            g_copy(1, isb, buf1, sem1).start()

            @pl.loop(0, CPS, step=2)
            def _(i, isb=isb, idb=idb):
                g_copy(i, isb, buf0, sem0).wait()
                s_copy(i, idb, buf0, sems0)
                g_copy(i + 1, isb, buf1, sem1).wait()
                s_copy(i + 1, idb, buf1, sems1)

                @pl.when(i + 2 < CPS)
                def _():
                    s_wait(i, idb, buf0, sems0)
                    g_copy(i + 2, isb, buf0, sem0).start()

                @pl.when(i + 3 < CPS)
                def _():
                    s_wait(i + 1, idb, buf1, sems1)
                    g_copy(i + 3, isb, buf1, sem1).start()

        s_wait(CPS - 2, banks[(SEG - 1) % 2][1], buf0, sems0)
        s_wait(CPS - 1, banks[(SEG - 1) % 2][1], buf1, sems1)
        plsc.subcore_barrier()
        pltpu.sync_copy(agg_sh.at[pl.ds(s * RPS, RPS)],
                        out_hbm.at[c].at[pl.ds(s * RPS, RPS)])

        @pl.when(s == NS - 1)
        def _():
            pltpu.sync_copy(agg_sh.at[pl.ds(NS * RPS, RREM)],
                            out_hbm.at[c].at[pl.ds(NS * RPS, RREM)])

    return k(h, edges4, zeros)


_BLK = 2048


def _lin1(x, Wl, bl):
    def body(x_ref, w_ref, b_ref, o_ref):
        o_ref[...] = jnp.dot(x_ref[...], w_ref[...],
                             preferred_element_type=jnp.float32) + b_ref[...]

    return pl.pallas_call(
        body,
        grid=(pl.cdiv(N, _BLK),),
        in_specs=[
            pl.BlockSpec((_BLK, D), lambda i: (i, 0)),
            pl.BlockSpec((D, D), lambda i: (0, 0)),
            pl.BlockSpec((1, D), lambda i: (0, 0)),
        ],
        out_specs=pl.BlockSpec((_BLK, D), lambda i: (i, 0)),
        out_shape=jax.ShapeDtypeStruct((N, D), jnp.float32),
    )(x, Wl, bl.reshape(1, D))


def _mlp(h, parts, W1, b1, W2, b2, W3, b3):
    def body(h_ref, p_ref, w1_ref, b1_ref, w2_ref, b2_ref, w3_ref, b3_ref,
             o_ref):
        z = h_ref[...] + p_ref[0] + p_ref[1]
        z = jnp.maximum(
            jnp.dot(z, w1_ref[...], preferred_element_type=jnp.float32)
            + b1_ref[...], 0.0)
        z = jnp.maximum(
            jnp.dot(z, w2_ref[...], preferred_element_type=jnp.float32)
            + b2_ref[...], 0.0)
        z = jnp.maximum(
            jnp.dot(z, w3_ref[...], preferred_element_type=jnp.float32)
            + b3_ref[...], 0.0)
        o_ref[...] = z

    return pl.pallas_call(
        body,
        grid=(pl.cdiv(N, _BLK),),
        in_specs=[
            pl.BlockSpec((_BLK, D), lambda i: (i, 0)),
            pl.BlockSpec((NC, _BLK, D), lambda i: (0, i, 0)),
            pl.BlockSpec((D, D), lambda i: (0, 0)),
            pl.BlockSpec((1, D), lambda i: (0, 0)),
            pl.BlockSpec((D, D), lambda i: (0, 0)),
            pl.BlockSpec((1, D), lambda i: (0, 0)),
            pl.BlockSpec((D, D), lambda i: (0, 0)),
            pl.BlockSpec((1, D), lambda i: (0, 0)),
        ],
        out_specs=pl.BlockSpec((_BLK, D), lambda i: (i, 0)),
        out_shape=jax.ShapeDtypeStruct((N, D), jnp.float32),
    )(h, parts, W1, b1.reshape(1, D), W2, b2.reshape(1, D), W3,
      b3.reshape(1, D))


def _pad_edges(edge_index):
    npw = EPW - E // NW
    pad_idx = jnp.arange(npw, dtype=jnp.int32)
    pad = jnp.broadcast_to(
        jnp.stack([pad_idx, N + pad_idx])[:, None, :], (2, NW, npw))
    real = edge_index.reshape(2, NW, E // NW)
    return jnp.concatenate([real, pad], axis=2).reshape(2, NW, NCH, CHUNK)


def kernel(x, edge_index, Wl, bl, W1, b1, W2, b2, W3, b3):
    edges4 = _pad_edges(edge_index)
    zeros = jnp.zeros((ZROWS, D), jnp.float32)
    h = _lin1(x, Wl, bl)
    parts = _sc_segment_partials(h, edges4, zeros)
    return _mlp(h, parts, W1, b1, W2, b2, W3, b3)


# speedup vs baseline: 1.1683x; 1.1683x over previous
import functools

import jax
import jax.numpy as jnp
from jax import lax
from jax.experimental import pallas as pl
from jax.experimental.pallas import tpu as pltpu
from jax.experimental.pallas import tpu_sc as plsc

N, E, D = 10000, 320000, 128
NC, NS = 2, 16
NW = NC * NS
CHUNK = 128
NCH = 80
SEG = 5
CPS = NCH // SEG
EPW = NCH * CHUNK
EPAD = NW * EPW
NPAD = 240
RPS = 624
RREM = N - NS * RPS


ZROWS = 640


def _sc_segment_partials(h, edges4, zeros):
    mesh = plsc.VectorSubcoreMesh(core_axis_name="c", subcore_axis_name="s")

    @functools.partial(
        pl.kernel,
        out_type=jax.ShapeDtypeStruct((NC, N, D), jnp.float32),
        mesh=mesh,
        scratch_types=[
            pltpu.VMEM((CPS, CHUNK), jnp.int32),
            pltpu.VMEM((CPS, CHUNK), jnp.int32),
            pltpu.VMEM((CPS, CHUNK), jnp.int32),
            pltpu.VMEM((CPS, CHUNK), jnp.int32),
            pltpu.VMEM((CHUNK, D), jnp.float32),
            pltpu.VMEM((CHUNK, D), jnp.float32),
            pltpu.VMEM_SHARED((N + NPAD, D), jnp.float32),
            pltpu.SemaphoreType.DMA,
            pltpu.SemaphoreType.DMA,
            pltpu.SemaphoreType.DMA,
            pltpu.SemaphoreType.DMA,
        ],
    )
    def k(h_hbm, e_hbm, zero_hbm, out_hbm,
          is0, id0, is1, id1, buf0, buf1, agg_sh,
          sem0, sem1, semi0, semi1):
        c = lax.axis_index("c")
        s = lax.axis_index("s")
        wid = c * NS + s

        def idx_copies(q, isb, idb, semi):
            qs = pl.ds(q * CPS, CPS)
            return (
                pltpu.make_async_copy(e_hbm.at[0].at[wid].at[qs], isb, semi),
                pltpu.make_async_copy(e_hbm.at[1].at[wid].at[qs], idb, semi),
            )

        def load_idx(q, isb, idb, semi):
            for cp in idx_copies(q, isb, idb, semi):
                cp.start()

        def wait_idx(q, isb, idb, semi):
            for cp in idx_copies(q, isb, idb, semi):
                cp.wait()

        load_idx(0, is0, id0, semi0)

        pltpu.sync_copy(zero_hbm.at[pl.ds(0, RPS)],
                        agg_sh.at[pl.ds(s * RPS, RPS)])

        @pl.when(s == NS - 1)
        def _():
            pltpu.sync_copy(zero_hbm.at[pl.ds(0, RREM + NPAD)],
                            agg_sh.at[pl.ds(NS * RPS, RREM + NPAD)])

        plsc.subcore_barrier()

        def g_copy(i, isb, buf, sem):
            return pltpu.make_async_copy(h_hbm.at[isb.at[i]], buf, sem)

        def scat(i, idb, buf):
            pltpu.sync_copy(buf, agg_sh.at[idb.at[i]], add=True)

        banks = ((is0, id0, semi0), (is1, id1, semi1))
        for q in range(SEG):
            isb, idb, semi = banks[q % 2]
            wait_idx(q, isb, idb, semi)
            if q + 1 < SEG:
                load_idx(q + 1, *banks[(q + 1) % 2])

            g_copy(0, isb, buf0, sem0).start()

            @pl.loop(0, CPS, step=2)
            def _(i, isb=isb, idb=idb):
                g_copy(i + 1, isb, buf1, sem1).start()
                g_copy(i, isb, buf0, sem0).wait()
                scat(i, idb, buf0)

                @pl.when(i + 2 < CPS)
                def _():
                    g_copy(i + 2, isb, buf0, sem0).start()

                g_copy(i + 1, isb, buf1, sem1).wait()
                scat(i + 1, idb, buf1)

        plsc.subcore_barrier()
        pltpu.sync_copy(agg_sh.at[pl.ds(s * RPS, RPS)],
                        out_hbm.at[c].at[pl.ds(s * RPS, RPS)])

        @pl.when(s == NS - 1)
        def _():
            pltpu.sync_copy(agg_sh.at[pl.ds(NS * RPS, RREM)],
                            out_hbm.at[c].at[pl.ds(NS * RPS, RREM)])

    return k(h, edges4, zeros)


_BLK = 1024


def _lin1(x, Wl, bl):
    def body(x_ref, w_ref, b_ref, o_ref):
        o_ref[...] = jnp.dot(x_ref[...], w_ref[...],
                             preferred_element_type=jnp.float32) + b_ref[...]

    return pl.pallas_call(
        body,
        grid=(pl.cdiv(N, _BLK),),
        in_specs=[
            pl.BlockSpec((_BLK, D), lambda i: (i, 0)),
            pl.BlockSpec((D, D), lambda i: (0, 0)),
            pl.BlockSpec((1, D), lambda i: (0, 0)),
        ],
        out_specs=pl.BlockSpec((_BLK, D), lambda i: (i, 0)),
        out_shape=jax.ShapeDtypeStruct((N, D), jnp.float32),
    )(x, Wl, bl.reshape(1, D))


def _mlp(h, parts, W1, b1, W2, b2, W3, b3):
    def body(h_ref, p_ref, w1_ref, b1_ref, w2_ref, b2_ref, w3_ref, b3_ref,
             o_ref):
        z = h_ref[...] + p_ref[0] + p_ref[1]
        z = jnp.maximum(
            jnp.dot(z, w1_ref[...], preferred_element_type=jnp.float32)
            + b1_ref[...], 0.0)
        z = jnp.maximum(
            jnp.dot(z, w2_ref[...], preferred_element_type=jnp.float32)
            + b2_ref[...], 0.0)
        z = jnp.maximum(
            jnp.dot(z, w3_ref[...], preferred_element_type=jnp.float32)
            + b3_ref[...], 0.0)
        o_ref[...] = z

    return pl.pallas_call(
        body,
        grid=(pl.cdiv(N, _BLK),),
        in_specs=[
            pl.BlockSpec((_BLK, D), lambda i: (i, 0)),
            pl.BlockSpec((NC, _BLK, D), lambda i: (0, i, 0)),
            pl.BlockSpec((D, D), lambda i: (0, 0)),
            pl.BlockSpec((1, D), lambda i: (0, 0)),
            pl.BlockSpec((D, D), lambda i: (0, 0)),
            pl.BlockSpec((1, D), lambda i: (0, 0)),
            pl.BlockSpec((D, D), lambda i: (0, 0)),
            pl.BlockSpec((1, D), lambda i: (0, 0)),
        ],
        out_specs=pl.BlockSpec((_BLK, D), lambda i: (i, 0)),
        out_shape=jax.ShapeDtypeStruct((N, D), jnp.float32),
    )(h, parts, W1, b1.reshape(1, D), W2, b2.reshape(1, D), W3,
      b3.reshape(1, D))


def _pad_edges(edge_index):
    npw = EPW - E // NW
    pad_idx = jnp.arange(npw, dtype=jnp.int32)
    pad = jnp.broadcast_to(
        jnp.stack([pad_idx, N + pad_idx])[:, None, :], (2, NW, npw))
    real = edge_index.reshape(2, NW, E // NW)
    return jnp.concatenate([real, pad], axis=2).reshape(2, NW, NCH, CHUNK)


def kernel(x, edge_index, Wl, bl, W1, b1, W2, b2, W3, b3):
    edges4 = _pad_edges(edge_index)
    zeros = jnp.zeros((ZROWS, D), jnp.float32)
    h = _lin1(x, Wl, bl)
    parts = _sc_segment_partials(h, edges4, zeros)
    return _mlp(h, parts, W1, b1, W2, b2, W3, b3)


# speedup vs baseline: 1.2032x; 1.0299x over previous
import functools

import jax
import jax.numpy as jnp
from jax import lax
from jax.experimental import pallas as pl
from jax.experimental.pallas import tpu as pltpu
from jax.experimental.pallas import tpu_sc as plsc

N, E, D = 10000, 320000, 128
NC, NS = 2, 16
NW = NC * NS
CHUNK = 128
NCH = 80
SEG = 5
CPS = NCH // SEG
EPW = NCH * CHUNK
EPAD = NW * EPW
NPAD = 240
RPS = 624
RREM = N - NS * RPS


ZROWS = 640


def _sc_segment_partials(h, edges4, zeros):
    mesh = plsc.VectorSubcoreMesh(core_axis_name="c", subcore_axis_name="s")

    @functools.partial(
        pl.kernel,
        out_type=jax.ShapeDtypeStruct((NC, N, D), jnp.float32),
        mesh=mesh,
        scratch_types=[
            pltpu.VMEM((CPS, CHUNK), jnp.int32),
            pltpu.VMEM((CPS, CHUNK), jnp.int32),
            pltpu.VMEM((CPS, CHUNK), jnp.int32),
            pltpu.VMEM((CPS, CHUNK), jnp.int32),
            pltpu.VMEM((CHUNK, D), jnp.float32),
            pltpu.VMEM((CHUNK, D), jnp.float32),
            pltpu.VMEM_SHARED((N + NPAD, D), jnp.float32),
            pltpu.SemaphoreType.DMA,
            pltpu.SemaphoreType.DMA,
            pltpu.SemaphoreType.DMA,
            pltpu.SemaphoreType.DMA,
        ],
    )
    def k(h_hbm, e_hbm, zero_hbm, out_hbm,
          is0, id0, is1, id1, buf0, buf1, agg_sh,
          sem0, sem1, semi0, semi1):
        c = lax.axis_index("c")
        s = lax.axis_index("s")
        wid = c * NS + s

        def idx_copies(q, isb, idb, semi):
            qs = pl.ds(q * CPS, CPS)
            return (
                pltpu.make_async_copy(e_hbm.at[0].at[wid].at[qs], isb, semi),
                pltpu.make_async_copy(e_hbm.at[1].at[wid].at[qs], idb, semi),
            )

        def load_idx(q, isb, idb, semi):
            for cp in idx_copies(q, isb, idb, semi):
                cp.start()

        def wait_idx(q, isb, idb, semi):
            for cp in idx_copies(q, isb, idb, semi):
                cp.wait()

        load_idx(0, is0, id0, semi0)

        pltpu.sync_copy(zero_hbm.at[pl.ds(0, RPS)],
                        agg_sh.at[pl.ds(s * RPS, RPS)])

        @pl.when(s == NS - 1)
        def _():
            pltpu.sync_copy(zero_hbm.at[pl.ds(0, RREM + NPAD)],
                            agg_sh.at[pl.ds(NS * RPS, RREM + NPAD)])

        plsc.subcore_barrier()

        def g_copy(i, isb, buf, sem):
            return pltpu.make_async_copy(h_hbm.at[isb.at[i]], buf, sem)

        def scat(i, idb, buf):
            pltpu.sync_copy(buf, agg_sh.at[idb.at[i]], add=True)

        banks = ((is0, id0, semi0), (is1, id1, semi1))
        for q in range(SEG):
            isb, idb, semi = banks[q % 2]
            wait_idx(q, isb, idb, semi)
            if q + 1 < SEG:
                load_idx(q + 1, *banks[(q + 1) % 2])

            g_copy(0, isb, buf0, sem0).start()

            @pl.loop(0, CPS, step=2)
            def _(i, isb=isb, idb=idb):
                g_copy(i + 1, isb, buf1, sem1).start()
                g_copy(i, isb, buf0, sem0).wait()
                scat(i, idb, buf0)

                @pl.when(i + 2 < CPS)
                def _():
                    g_copy(i + 2, isb, buf0, sem0).start()

                g_copy(i + 1, isb, buf1, sem1).wait()
                scat(i + 1, idb, buf1)

        plsc.subcore_barrier()
        pltpu.sync_copy(agg_sh.at[pl.ds(s * RPS, RPS)],
                        out_hbm.at[c].at[pl.ds(s * RPS, RPS)])

        @pl.when(s == NS - 1)
        def _():
            pltpu.sync_copy(agg_sh.at[pl.ds(NS * RPS, RREM)],
                            out_hbm.at[c].at[pl.ds(NS * RPS, RREM)])

    return k(h, edges4, zeros)


_BLK = 2048


def _lin1(x, Wl, bl):
    def body(x_ref, w_ref, b_ref, o_ref):
        o_ref[...] = jnp.dot(x_ref[...], w_ref[...],
                             preferred_element_type=jnp.float32) + b_ref[...]

    return pl.pallas_call(
        body,
        grid=(pl.cdiv(N, _BLK),),
        in_specs=[
            pl.BlockSpec((_BLK, D), lambda i: (i, 0)),
            pl.BlockSpec((D, D), lambda i: (0, 0)),
            pl.BlockSpec((1, D), lambda i: (0, 0)),
        ],
        out_specs=pl.BlockSpec((_BLK, D), lambda i: (i, 0)),
        out_shape=jax.ShapeDtypeStruct((N, D), jnp.float32),
    )(x, Wl, bl.reshape(1, D))


def _mlp(h, parts, W1, b1, W2, b2, W3, b3):
    def body(h_ref, p_ref, w1_ref, b1_ref, w2_ref, b2_ref, w3_ref, b3_ref,
             o_ref):
        z = h_ref[...] + p_ref[0] + p_ref[1]
        z = jnp.maximum(
            jnp.dot(z, w1_ref[...], preferred_element_type=jnp.float32)
            + b1_ref[...], 0.0)
        z = jnp.maximum(
            jnp.dot(z, w2_ref[...], preferred_element_type=jnp.float32)
            + b2_ref[...], 0.0)
        z = jnp.maximum(
            jnp.dot(z, w3_ref[...], preferred_element_type=jnp.float32)
            + b3_ref[...], 0.0)
        o_ref[...] = z

    return pl.pallas_call(
        body,
        grid=(pl.cdiv(N, _BLK),),
        in_specs=[
            pl.BlockSpec((_BLK, D), lambda i: (i, 0)),
            pl.BlockSpec((NC, _BLK, D), lambda i: (0, i, 0)),
            pl.BlockSpec((D, D), lambda i: (0, 0)),
            pl.BlockSpec((1, D), lambda i: (0, 0)),
            pl.BlockSpec((D, D), lambda i: (0, 0)),
            pl.BlockSpec((1, D), lambda i: (0, 0)),
            pl.BlockSpec((D, D), lambda i: (0, 0)),
            pl.BlockSpec((1, D), lambda i: (0, 0)),
        ],
        out_specs=pl.BlockSpec((_BLK, D), lambda i: (i, 0)),
        out_shape=jax.ShapeDtypeStruct((N, D), jnp.float32),
    )(h, parts, W1, b1.reshape(1, D), W2, b2.reshape(1, D), W3,
      b3.reshape(1, D))


def _pad_edges(edge_index):
    npw = EPW - E // NW
    pad_idx = jnp.arange(npw, dtype=jnp.int32)
    pad = jnp.broadcast_to(
        jnp.stack([pad_idx, N + pad_idx])[:, None, :], (2, NW, npw))
    real = edge_index.reshape(2, NW, E // NW)
    return jnp.concatenate([real, pad], axis=2).reshape(2, NW, NCH, CHUNK)


def kernel(x, edge_index, Wl, bl, W1, b1, W2, b2, W3, b3):
    edges4 = _pad_edges(edge_index)
    zeros = jnp.zeros((ZROWS, D), jnp.float32)
    h = _lin1(x, Wl, bl)
    parts = _sc_segment_partials(h, edges4, zeros)
    return _mlp(h, parts, W1, b1, W2, b2, W3, b3)
